# baseline (speedup 1.0000x reference)
import jax
import jax.numpy as jnp
from jax import lax
from jax.experimental import pallas as pl
from jax.experimental.pallas import tpu as pltpu
from jax.experimental.pallas import tpu_sc as plsc

N = 10000
E = 320000
HID = 128
OUT_CH = 10
NUM_GRAPHS = 64
NEG_SLOPE = 0.2

NPAD = 10240
DUMMY = N
K = 112
NW = 32
EDGES = E + N
CPB = 32
NBLK = 3
CHUNKS = CPB * NBLK
PW = K * CHUNKS
EP = NW * PW
BSZ = K * CPB
HALF = NPAD // 2
ACC_ROWS = 5376
RPT = ACC_ROWS // 16
HPT = HALF // 16


def _xform_body(x_ref, wl_ref, bl_ref, wr_ref, br_ref, xl_ref, xr_ref):
    x = x_ref[...]
    xl_ref[...] = jnp.dot(x, wl_ref[...], preferred_element_type=jnp.float32) + bl_ref[...]
    xr_ref[...] = jnp.dot(x, wr_ref[...], preferred_element_type=jnp.float32) + br_ref[...]


def _sc_edge_kernel(xl_hbm, xr_hbm, src_hbm, dst_hbm, att_hbm,
                    numer_hbm, denom_hbm,
                    src_blk, dst_blk,
                    isrc0, isrc1, idst0, idst1, iloc0, iloc1,
                    xl0, xl1, xr0, xr1,
                    accbuf, kbuf, att_v, wstore, denom_t, acc_sh,
                    sg0, sg1, ss0, ss1):
    c = lax.axis_index("c")
    s = lax.axis_index("s")
    wid = s * 2 + c

    isrc = [isrc0, isrc1]
    idst = [idst0, idst1]
    iloc = [iloc0, iloc1]
    xls = [xl0, xl1]
    xrs = [xr0, xr1]
    outs = xrs
    sg = [sg0, sg1]
    ss = [ss0, ss1]
    out0 = xr0

    pltpu.sync_copy(att_hbm, att_v)
    att_vecs = [att_v[pl.ds(j * 16, 16)] for j in range(8)]
    iota = lax.iota(jnp.int32, 16)
    spread = HALF + (iota & 7)
    zeros16 = jnp.zeros((16,), jnp.float32)

    def _zero_den(i, _):
        denom_t[pl.ds(i * 16, 16)] = zeros16
        return 0
    lax.fori_loop(0, NPAD // 16, _zero_den, 0)

    def _zero_acc():
        def _zero_row(i, _):
            for j in range(HID // 16):
                out0[i, pl.ds(j * 16, 16)] = zeros16
            return 0
        lax.fori_loop(0, K, _zero_row, 0)
        for r0 in (0, K, 2 * K):
            pltpu.sync_copy(out0, acc_sh.at[pl.ds(s * RPT + r0, K)])

    def _dump_half(row_base):
        for r0, n in ((0, K), (K, K), (2 * K, HPT - 2 * K)):
            r = s * HPT + r0
            pltpu.sync_copy(acc_sh.at[pl.ds(r, n)], out0.at[pl.ds(0, n)])
            pltpu.sync_copy(out0.at[pl.ds(0, n)],
                            numer_hbm.at[c, pl.ds(row_base + r, n)])

    def _stage_idx(ci, b, pass1):
        for j in range(K // 16):
            o = pl.ds(j * 16, 16)
            isrc[b][o] = src_blk[pl.ds(ci * K + j * 16, 16)]
            idst[b][o] = dst_blk[pl.ds(ci * K + j * 16, 16)]

    def _fire_gathers(b, pass1):
        pltpu.async_copy(xl_hbm.at[isrc[b]], xls[b], sg[b])
        if pass1:
            pltpu.async_copy(xr_hbm.at[idst[b]], xrs[b], sg[b])

    def _wait_gathers(b, pass1):
        pltpu.make_async_copy(xl_hbm.at[isrc[b]], xls[b], sg[b]).wait()
        if pass1:
            pltpu.make_async_copy(xr_hbm.at[idst[b]], xrs[b], sg[b]).wait()

    def _fire_scatter(b):
        pltpu.async_copy(outs[b], acc_sh.at[iloc[b]], ss[b], add=True)

    def _wait_scatter(b):
        pltpu.make_async_copy(outs[b], acc_sh.at[iloc[b]], ss[b]).wait()

    def _compute(blk, ci, b, pass1):
        xl_rows, xr_rows, out_rows = xls[b], xrs[b], outs[b]

        def group(g, _):
            gb = pl.multiple_of(g * 16, 16)
            if pass1:
                for e in range(16):
                    acc = zeros16
                    for j in range(8):
                        u = (xl_rows[gb + e, pl.ds(j * 16, 16)]
                             + xr_rows[gb + e, pl.ds(j * 16, 16)])
                        lr = jnp.where(u >= 0.0, u, u * NEG_SLOPE)
                        acc = acc + att_vecs[j] * lr
                    accbuf[pl.ds(e * 16, 16)] = acc
                esum = zeros16
                for d in range(16):
                    esum = esum + plsc.load_gather(accbuf, [iota * 16 + d])
                w = jnp.exp(esum)
                wstore[pl.ds((blk * CPB + ci) * K + gb, 16)] = w
            else:
                w = wstore[pl.ds((blk * CPB + ci) * K + gb, 16)]
            for e in range(16):
                ws = w[e]
                for j in range(HID // 16):
                    out_rows[gb + e, pl.ds(j * 16, 16)] = (
                        xl_rows[gb + e, pl.ds(j * 16, 16)] * ws)
            dst16 = idst[b][pl.ds(gb, 16)]
            if pass1:
                iloc[b][pl.ds(gb, 16)] = jnp.where(dst16 < HALF, dst16,
                                                   spread)
                kd, vw = plsc.sort_key_val(dst16, w)
                cum = plsc.cumsum(vw)
                kbuf[...] = kd
                knext = plsc.load_gather(kbuf, [jnp.minimum(iota + 1, 15)])
                last = (kd != knext) | (iota == 15)
                plsc.addupdate_scatter(denom_t, [kd], cum, mask=last)
                plsc.addupdate_scatter(denom_t, [knext], -cum,
                                       mask=last & (iota < 15))
            else:
                iloc[b][pl.ds(gb, 16)] = jnp.where(dst16 >= HALF,
                                                   dst16 - HALF, spread)
            return 0

        lax.fori_loop(0, K // 16, group, 0)

    def _do_pass(pass1):
        for blk in range(NBLK):
            bbase = wid * PW + blk * BSZ
            pltpu.sync_copy(src_hbm.at[pl.ds(bbase, BSZ)], src_blk)
            pltpu.sync_copy(dst_hbm.at[pl.ds(bbase, BSZ)], dst_blk)
            def pair(ii, _):
                for b in range(2):
                    ci = ii * 2 + b
                    _stage_idx(ci, b, pass1)
                    _fire_gathers(b, pass1)
                    _wait_gathers(b, pass1)
                    _compute(blk, ci, b, pass1)
                    pltpu.sync_copy(outs[b], acc_sh.at[iloc[b]], add=True)
                return 0

            lax.fori_loop(0, CPB // 2, pair, 0)

    _zero_acc()
    plsc.subcore_barrier()
    _do_pass(True)
    plsc.subcore_barrier()
    _dump_half(0)
    plsc.subcore_barrier()
    _zero_acc()
    plsc.subcore_barrier()
    _do_pass(False)
    plsc.subcore_barrier()
    _dump_half(HALF)
    pltpu.sync_copy(denom_t, denom_hbm.at[wid])


def _sc_edge_call(xl, xr, src, dst, att):
    mesh = plsc.VectorSubcoreMesh(core_axis_name="c", subcore_axis_name="s")
    return pl.kernel(
        _sc_edge_kernel,
        out_type=(jax.ShapeDtypeStruct((2, NPAD, HID), jnp.float32),
                  jax.ShapeDtypeStruct((NW, NPAD), jnp.float32)),
        mesh=mesh,
        compiler_params=pltpu.CompilerParams(needs_layout_passes=False),
        scratch_types=[
            pltpu.VMEM((BSZ,), jnp.int32),
            pltpu.VMEM((BSZ,), jnp.int32),
            pltpu.VMEM((K,), jnp.int32),
            pltpu.VMEM((K,), jnp.int32),
            pltpu.VMEM((K,), jnp.int32),
            pltpu.VMEM((K,), jnp.int32),
            pltpu.VMEM((K,), jnp.int32),
            pltpu.VMEM((K,), jnp.int32),
            pltpu.VMEM((K, HID), jnp.float32),
            pltpu.VMEM((K, HID), jnp.float32),
            pltpu.VMEM((K, HID), jnp.float32),
            pltpu.VMEM((K, HID), jnp.float32),
            pltpu.VMEM((256,), jnp.float32),
            pltpu.VMEM((16,), jnp.int32),
            pltpu.VMEM((HID,), jnp.float32),
            pltpu.VMEM((PW,), jnp.float32),
            pltpu.VMEM((NPAD,), jnp.float32),
            pltpu.VMEM_SHARED((ACC_ROWS, HID), jnp.float32),
            pltpu.SemaphoreType.DMA,
            pltpu.SemaphoreType.DMA,
            pltpu.SemaphoreType.DMA,
            pltpu.SemaphoreType.DMA,
        ],
    )(xl, xr, src, dst, att)


def _final_body(numer_ref, denom_ref, bias_ref, batch_ref, wlin_ref, blin_ref,
                out_ref):
    numer = numer_ref[0] + numer_ref[1]
    denom = jnp.sum(denom_ref[...], axis=0)[:, None]
    h = jnp.maximum(numer / (denom + 1e-16) + bias_ref[...], 0.0)
    b = batch_ref[...]
    g = lax.broadcasted_iota(jnp.int32, (NUM_GRAPHS, NPAD), 0)
    m = (b == g).astype(jnp.float32)
    sums = jnp.dot(m, h, preferred_element_type=jnp.float32)
    counts = jnp.sum(m, axis=1, keepdims=True)
    pooled = sums / jnp.clip(counts, 1.0, None)
    logits = jnp.dot(pooled, wlin_ref[...], preferred_element_type=jnp.float32)
    logits = logits + blin_ref[...]
    mx = jnp.max(logits, axis=1, keepdims=True)
    z = logits - mx
    out_ref[...] = z - jnp.log(jnp.sum(jnp.exp(z), axis=1, keepdims=True))


def kernel(x, edge_index, batch, Wl, bl, Wr, br, att, bias, Wlin, blin):
    loops = jnp.arange(N, dtype=edge_index.dtype)
    pad = jnp.full((EP - EDGES,), DUMMY, dtype=edge_index.dtype)
    src = jnp.concatenate([edge_index[0], loops, pad])
    dst = jnp.concatenate([edge_index[1], loops, pad])
    x_pad = jnp.pad(x, ((0, NPAD - N), (0, 0)))

    xl, xr = pl.pallas_call(
        _xform_body,
        out_shape=(jax.ShapeDtypeStruct((NPAD, HID), jnp.float32),
                   jax.ShapeDtypeStruct((NPAD, HID), jnp.float32)),
    )(x_pad, Wl, bl.reshape(1, -1), Wr, br.reshape(1, -1))

    numer, denom = _sc_edge_call(xl, xr, src, dst, att)

    batch_pad = jnp.pad(batch, (0, NPAD - N), constant_values=NUM_GRAPHS)
    return pl.pallas_call(
        _final_body,
        out_shape=jax.ShapeDtypeStruct((NUM_GRAPHS, OUT_CH), jnp.float32),
    )(numer, denom, bias.reshape(1, -1), batch_pad.reshape(1, -1),
      Wlin, blin.reshape(1, -1))


# speedup vs baseline: 1.8812x; 1.8812x over previous
import jax
import jax.numpy as jnp
from jax import lax
from jax.experimental import pallas as pl
from jax.experimental.pallas import tpu as pltpu
from jax.experimental.pallas import tpu_sc as plsc

N = 10000
E = 320000
HID = 128
OUT_CH = 10
NUM_GRAPHS = 64
NEG_SLOPE = 0.2

NPAD = 10240
DUMMY = N
K = 80
NW = 32
EDGES = E + N
CHUNKS = 132
PW = K * CHUNKS
EP = NW * PW
HALF = NPAD // 2
ACC_ROWS = 5376
RPT = ACC_ROWS // 16
HPT = HALF // 16


def _xform_body(x_ref, wl_ref, bl_ref, wr_ref, br_ref, xl_ref, xr_ref):
    x = x_ref[...]
    xl_ref[...] = jnp.dot(x, wl_ref[...], preferred_element_type=jnp.float32) + bl_ref[...]
    xr_ref[...] = jnp.dot(x, wr_ref[...], preferred_element_type=jnp.float32) + br_ref[...]


def _sc_edge_kernel(xl_hbm, xr_hbm, src_hbm, dst_hbm, att_hbm,
                    numer_hbm, denom_hbm,
                    isrc0, isrc1, idst0, idst1, iloc0,
                    xl0, xl1, xr0, xr1, outb0, outb1,
                    accbuf, kbuf, att_v, wstore, denom_t, acc_sh,
                    sg0, sg1, si0, si1):
    c = lax.axis_index("c")
    s = lax.axis_index("s")
    wid = s * 2 + c

    isrc = [isrc0, isrc1]
    idst = [idst0, idst1]
    iloc = [iloc0, iloc0]
    xls = [xl0, xl1]
    xrs = [xr0, xr1]
    outs = [outb0, outb1]
    sg = [sg0, sg1]
    si = [si0, si1]
    out0 = outb0

    pltpu.sync_copy(att_hbm, att_v)
    att_vecs = [att_v[pl.ds(j * 16, 16)] for j in range(8)]
    iota = lax.iota(jnp.int32, 16)
    spread = HALF + (iota & 7)
    zeros16 = jnp.zeros((16,), jnp.float32)

    def _zero_den(i, _):
        denom_t[pl.ds(i * 16, 16)] = zeros16
        return 0
    lax.fori_loop(0, NPAD // 16, _zero_den, 0)

    def _zero_acc():
        def _zero_row(i, _):
            for j in range(HID // 16):
                out0[i, pl.ds(j * 16, 16)] = zeros16
            return 0
        lax.fori_loop(0, K, _zero_row, 0)
        for r0 in (0, K, 2 * K):
            pltpu.sync_copy(out0, acc_sh.at[pl.ds(s * RPT + r0, K)])

    def _dump_half(row_base):
        for r0, n in ((0, K), (K, K), (2 * K, HPT - 2 * K)):
            r = s * HPT + r0
            pltpu.sync_copy(acc_sh.at[pl.ds(r, n)], out0.at[pl.ds(0, n)])
            pltpu.sync_copy(out0.at[pl.ds(0, n)],
                            numer_hbm.at[c, pl.ds(row_base + r, n)])

    def _fire_idx(base, ci, b):
        pltpu.async_copy(src_hbm.at[pl.ds(base + ci * K, K)], isrc[b], si[b])
        pltpu.async_copy(dst_hbm.at[pl.ds(base + ci * K, K)], idst[b], si[b])

    def _wait_idx(base, ci, b):
        pltpu.make_async_copy(src_hbm.at[pl.ds(base + ci * K, K)], isrc[b],
                              si[b]).wait()
        pltpu.make_async_copy(dst_hbm.at[pl.ds(base + ci * K, K)], idst[b],
                              si[b]).wait()

    def _fire_gathers(b, pass1):
        pltpu.async_copy(xl_hbm.at[isrc[b]], xls[b], sg[b])
        if pass1:
            pltpu.async_copy(xr_hbm.at[idst[b]], xrs[b], sg[b])

    def _wait_gathers(b, pass1):
        pltpu.make_async_copy(xl_hbm.at[isrc[b]], xls[b], sg[b]).wait()
        if pass1:
            pltpu.make_async_copy(xr_hbm.at[idst[b]], xrs[b], sg[b]).wait()

    def _compute(ci, b, pass1):
        xl_rows, xr_rows, out_rows = xls[b], xrs[b], outs[b]

        def group(g, _):
            gb = pl.multiple_of(g * 16, 16)
            if pass1:
                for e in range(16):
                    acc = zeros16
                    for j in range(8):
                        u = (xl_rows[gb + e, pl.ds(j * 16, 16)]
                             + xr_rows[gb + e, pl.ds(j * 16, 16)])
                        lr = jnp.where(u >= 0.0, u, u * NEG_SLOPE)
                        acc = acc + att_vecs[j] * lr
                    accbuf[pl.ds(e * 16, 16)] = acc
                esum = zeros16
                for d in range(16):
                    esum = esum + plsc.load_gather(accbuf, [iota * 16 + d])
                w = jnp.exp(esum)
                wstore[pl.ds(ci * K + gb, 16)] = w
            else:
                w = wstore[pl.ds(ci * K + gb, 16)]
            for e in range(16):
                ws = w[e]
                for j in range(HID // 16):
                    out_rows[gb + e, pl.ds(j * 16, 16)] = (
                        xl_rows[gb + e, pl.ds(j * 16, 16)] * ws)
            dst16 = idst[b][pl.ds(gb, 16)]
            if pass1:
                iloc[b][pl.ds(gb, 16)] = jnp.where(dst16 < HALF, dst16,
                                                   spread)
                kd, vw = plsc.sort_key_val(dst16, w)
                cum = plsc.cumsum(vw)
                kbuf[...] = kd
                knext = plsc.load_gather(kbuf, [jnp.minimum(iota + 1, 15)])
                last = (kd != knext) | (iota == 15)
                plsc.addupdate_scatter(denom_t, [kd], cum, mask=last)
                plsc.addupdate_scatter(denom_t, [knext], -cum,
                                       mask=last & (iota < 15))
            else:
                iloc[b][pl.ds(gb, 16)] = jnp.where(dst16 >= HALF,
                                                   dst16 - HALF, spread)
            return 0

        lax.fori_loop(0, K // 16, group, 0)

    def _do_pass(pass1):
        base = wid * PW
        _fire_idx(base, 0, 0)
        _wait_idx(base, 0, 0)
        _fire_gathers(0, pass1)
        _fire_idx(base, 1, 1)

        def pair(ii, _):
            for b in range(2):
                ci = ii * 2 + b

                @pl.when(ci + 1 < CHUNKS)
                def _():
                    _wait_idx(base, ci + 1, 1 - b)
                    _fire_gathers(1 - b, pass1)

                _wait_gathers(b, pass1)
                _compute(ci, b, pass1)

                @pl.when(ci + 2 < CHUNKS)
                def _():
                    _fire_idx(base, ci + 2, b)

                pltpu.sync_copy(outs[b], acc_sh.at[iloc[b]], add=True)
            return 0

        lax.fori_loop(0, CHUNKS // 2, pair, 0)

    _zero_acc()
    plsc.subcore_barrier()
    _do_pass(True)
    plsc.subcore_barrier()
    _dump_half(0)
    plsc.subcore_barrier()
    _zero_acc()
    plsc.subcore_barrier()
    _do_pass(False)
    plsc.subcore_barrier()
    _dump_half(HALF)
    pltpu.sync_copy(denom_t, denom_hbm.at[wid])


def _sc_edge_call(xl, xr, src, dst, att):
    mesh = plsc.VectorSubcoreMesh(core_axis_name="c", subcore_axis_name="s")
    return pl.kernel(
        _sc_edge_kernel,
        out_type=(jax.ShapeDtypeStruct((2, NPAD, HID), jnp.float32),
                  jax.ShapeDtypeStruct((NW, NPAD), jnp.float32)),
        mesh=mesh,
        compiler_params=pltpu.CompilerParams(needs_layout_passes=False),
        scratch_types=[
            pltpu.VMEM((K,), jnp.int32),
            pltpu.VMEM((K,), jnp.int32),
            pltpu.VMEM((K,), jnp.int32),
            pltpu.VMEM((K,), jnp.int32),
            pltpu.VMEM((K,), jnp.int32),
            pltpu.VMEM((K, HID), jnp.float32),
            pltpu.VMEM((K, HID), jnp.float32),
            pltpu.VMEM((K, HID), jnp.float32),
            pltpu.VMEM((K, HID), jnp.float32),
            pltpu.VMEM((K, HID), jnp.float32),
            pltpu.VMEM((K, HID), jnp.float32),
            pltpu.VMEM((256,), jnp.float32),
            pltpu.VMEM((16,), jnp.int32),
            pltpu.VMEM((HID,), jnp.float32),
            pltpu.VMEM((PW,), jnp.float32),
            pltpu.VMEM((NPAD,), jnp.float32),
            pltpu.VMEM_SHARED((ACC_ROWS, HID), jnp.float32),
            pltpu.SemaphoreType.DMA,
            pltpu.SemaphoreType.DMA,
            pltpu.SemaphoreType.DMA,
            pltpu.SemaphoreType.DMA,
        ],
    )(xl, xr, src, dst, att)


def _final_body(numer_ref, denom_ref, bias_ref, batch_ref, wlin_ref, blin_ref,
                out_ref):
    numer = numer_ref[0] + numer_ref[1]
    denom = jnp.sum(denom_ref[...], axis=0)[:, None]
    h = jnp.maximum(numer / (denom + 1e-16) + bias_ref[...], 0.0)
    b = batch_ref[...]
    g = lax.broadcasted_iota(jnp.int32, (NUM_GRAPHS, NPAD), 0)
    m = (b == g).astype(jnp.float32)
    sums = jnp.dot(m, h, preferred_element_type=jnp.float32)
    counts = jnp.sum(m, axis=1, keepdims=True)
    pooled = sums / jnp.clip(counts, 1.0, None)
    logits = jnp.dot(pooled, wlin_ref[...], preferred_element_type=jnp.float32)
    logits = logits + blin_ref[...]
    mx = jnp.max(logits, axis=1, keepdims=True)
    z = logits - mx
    out_ref[...] = z - jnp.log(jnp.sum(jnp.exp(z), axis=1, keepdims=True))


def kernel(x, edge_index, batch, Wl, bl, Wr, br, att, bias, Wlin, blin):
    loops = jnp.arange(N, dtype=edge_index.dtype)
    pad = jnp.full((EP - EDGES,), DUMMY, dtype=edge_index.dtype)
    src = jnp.concatenate([edge_index[0], loops, pad])
    dst = jnp.concatenate([edge_index[1], loops, pad])
    x_pad = jnp.pad(x, ((0, NPAD - N), (0, 0)))

    xl, xr = pl.pallas_call(
        _xform_body,
        out_shape=(jax.ShapeDtypeStruct((NPAD, HID), jnp.float32),
                   jax.ShapeDtypeStruct((NPAD, HID), jnp.float32)),
    )(x_pad, Wl, bl.reshape(1, -1), Wr, br.reshape(1, -1))

    numer, denom = _sc_edge_call(xl, xr, src, dst, att)

    batch_pad = jnp.pad(batch, (0, NPAD - N), constant_values=NUM_GRAPHS)
    return pl.pallas_call(
        _final_body,
        out_shape=jax.ShapeDtypeStruct((NUM_GRAPHS, OUT_CH), jnp.float32),
    )(numer, denom, bias.reshape(1, -1), batch_pad.reshape(1, -1),
      Wlin, blin.reshape(1, -1))


# speedup vs baseline: 2.0382x; 1.0835x over previous
import jax
import jax.numpy as jnp
from jax import lax
from jax.experimental import pallas as pl
from jax.experimental.pallas import tpu as pltpu
from jax.experimental.pallas import tpu_sc as plsc

N = 10000
E = 320000
HID = 128
OUT_CH = 10
NUM_GRAPHS = 64
NEG_SLOPE = 0.2

NPAD = 10240
DUMMY = N
K = 128
NW = 32
EDGES = E + N
CHUNKS = -(-EDGES // (NW * K))
EP = NW * K * CHUNKS
PW = K * CHUNKS
HALF = NPAD // 2
ACC_ROWS = 5376
RPT = ACC_ROWS // 16
HPT = HALF // 16


def _xform_body(x_ref, wl_ref, bl_ref, wr_ref, br_ref, xl_ref, xr_ref):
    x = x_ref[...]
    xl_ref[...] = jnp.dot(x, wl_ref[...], preferred_element_type=jnp.float32) + bl_ref[...]
    xr_ref[...] = jnp.dot(x, wr_ref[...], preferred_element_type=jnp.float32) + br_ref[...]


def _sc_edge_kernel(xl_hbm, xr_hbm, edges_hbm, att_hbm,
                    numer_hbm, denom_hbm,
                    idx2, idx_loc, xl_rows, xr_rows, out_rows,
                    accbuf, kbuf, att_v, wstore, denom_t, acc_sh,
                    sem_l, sem_r):
    c = lax.axis_index("c")
    s = lax.axis_index("s")
    wid = s * 2 + c

    pltpu.sync_copy(att_hbm, att_v)
    att_vecs = [att_v[pl.ds(j * 16, 16)] for j in range(8)]
    iota = lax.iota(jnp.int32, 16)
    spread = HALF + (iota & 7)
    zeros16 = jnp.zeros((16,), jnp.float32)

    def _zero_den(i, _):
        denom_t[pl.ds(i * 16, 16)] = zeros16
        return 0
    lax.fori_loop(0, NPAD // 16, _zero_den, 0)

    def _zero_acc():
        def _zero_row(i, _):
            for j in range(HID // 16):
                out_rows[i, pl.ds(j * 16, 16)] = zeros16
            return 0
        lax.fori_loop(0, K, _zero_row, 0)
        for r0 in (0, 128, 256):
            n = min(128, RPT - r0)
            pltpu.sync_copy(out_rows.at[pl.ds(0, n)],
                            acc_sh.at[pl.ds(s * RPT + r0, n)])

    def _dump_half(row_base):
        for r0 in (0, 128, 256):
            n = min(128, HPT - r0)
            r = s * HPT + r0
            pltpu.sync_copy(acc_sh.at[pl.ds(r, n)], out_rows.at[pl.ds(0, n)])
            pltpu.sync_copy(out_rows.at[pl.ds(0, n)],
                            numer_hbm.at[c, pl.ds(row_base + r, n)])

    _zero_acc()
    plsc.subcore_barrier()

    def chunk_body1(ci, carry):
        base = wid * PW + ci * K
        pltpu.sync_copy(edges_hbm.at[:, pl.ds(base, K)], idx2)
        cp_l = pltpu.async_copy(xl_hbm.at[idx2.at[0]], xl_rows, sem_l)
        cp_r = pltpu.async_copy(xr_hbm.at[idx2.at[1]], xr_rows, sem_r)
        cp_l.wait()
        cp_r.wait()

        def group(g, _):
            b = pl.multiple_of(g * 16, 16)
            for e in range(16):
                acc = zeros16
                for j in range(8):
                    u = (xl_rows[b + e, pl.ds(j * 16, 16)]
                         + xr_rows[b + e, pl.ds(j * 16, 16)])
                    lr = jnp.where(u >= 0.0, u, u * NEG_SLOPE)
                    acc = acc + att_vecs[j] * lr
                accbuf[pl.ds(e * 16, 16)] = acc
            esum = zeros16
            for d in range(16):
                esum = esum + plsc.load_gather(accbuf, [iota * 16 + d])
            w = jnp.exp(esum)
            wstore[pl.ds(ci * K + b, 16)] = w
            for e in range(16):
                ws = w[e]
                for j in range(HID // 16):
                    out_rows[b + e, pl.ds(j * 16, 16)] = (
                        xl_rows[b + e, pl.ds(j * 16, 16)] * ws)
            dst16 = idx2[1, pl.ds(b, 16)]
            idx_loc[pl.ds(b, 16)] = jnp.where(dst16 < HALF, dst16, spread)
            kd, vw = plsc.sort_key_val(dst16, w)
            cum = plsc.cumsum(vw)
            kbuf[...] = kd
            knext = plsc.load_gather(kbuf, [jnp.minimum(iota + 1, 15)])
            last = (kd != knext) | (iota == 15)
            plsc.addupdate_scatter(denom_t, [kd], cum, mask=last)
            plsc.addupdate_scatter(denom_t, [knext], -cum,
                                   mask=last & (iota < 15))
            return 0

        lax.fori_loop(0, K // 16, group, 0)
        pltpu.sync_copy(out_rows, acc_sh.at[idx_loc], add=True)
        return carry

    lax.fori_loop(0, CHUNKS, chunk_body1, 0)
    plsc.subcore_barrier()
    _dump_half(0)
    plsc.subcore_barrier()
    _zero_acc()
    plsc.subcore_barrier()

    def chunk_body2(ci, carry):
        base = wid * PW + ci * K
        pltpu.sync_copy(edges_hbm.at[:, pl.ds(base, K)], idx2)
        cp_l = pltpu.async_copy(xl_hbm.at[idx2.at[0]], xl_rows, sem_l)
        cp_l.wait()

        def group(g, _):
            b = pl.multiple_of(g * 16, 16)
            w = wstore[pl.ds(ci * K + b, 16)]
            for e in range(16):
                ws = w[e]
                for j in range(HID // 16):
                    out_rows[b + e, pl.ds(j * 16, 16)] = (
                        xl_rows[b + e, pl.ds(j * 16, 16)] * ws)
            dst16 = idx2[1, pl.ds(b, 16)]
            idx_loc[pl.ds(b, 16)] = jnp.where(dst16 >= HALF, dst16 - HALF,
                                              spread)
            return 0

        lax.fori_loop(0, K // 16, group, 0)
        pltpu.sync_copy(out_rows, acc_sh.at[idx_loc], add=True)
        return carry

    lax.fori_loop(0, CHUNKS, chunk_body2, 0)
    plsc.subcore_barrier()
    _dump_half(HALF)
    pltpu.sync_copy(denom_t, denom_hbm.at[wid])


def _sc_edge_call(xl, xr, edges, att):
    mesh = plsc.VectorSubcoreMesh(core_axis_name="c", subcore_axis_name="s")
    return pl.kernel(
        _sc_edge_kernel,
        out_type=(jax.ShapeDtypeStruct((2, NPAD, HID), jnp.float32),
                  jax.ShapeDtypeStruct((NW, NPAD), jnp.float32)),
        mesh=mesh,
        compiler_params=pltpu.CompilerParams(needs_layout_passes=False),
        scratch_types=[
            pltpu.VMEM((2, K), jnp.int32),
            pltpu.VMEM((K,), jnp.int32),
            pltpu.VMEM((K, HID), jnp.float32),
            pltpu.VMEM((K, HID), jnp.float32),
            pltpu.VMEM((K, HID), jnp.float32),
            pltpu.VMEM((256,), jnp.float32),
            pltpu.VMEM((16,), jnp.int32),
            pltpu.VMEM((HID,), jnp.float32),
            pltpu.VMEM((PW,), jnp.float32),
            pltpu.VMEM((NPAD,), jnp.float32),
            pltpu.VMEM_SHARED((ACC_ROWS, HID), jnp.float32),
            pltpu.SemaphoreType.DMA,
            pltpu.SemaphoreType.DMA,
        ],
    )(xl, xr, edges, att)


def _final_body(numer_ref, denom_ref, bias_ref, batch_ref, wlin_ref, blin_ref,
                out_ref):
    numer = numer_ref[0] + numer_ref[1]
    denom = jnp.sum(denom_ref[...], axis=0)[:, None]
    h = jnp.maximum(numer / (denom + 1e-16) + bias_ref[...], 0.0)
    b = batch_ref[...]
    g = lax.broadcasted_iota(jnp.int32, (NUM_GRAPHS, NPAD), 0)
    m = (b == g).astype(jnp.float32)
    sums = jnp.dot(m, h, preferred_element_type=jnp.float32)
    counts = jnp.sum(m, axis=1, keepdims=True)
    pooled = sums / jnp.clip(counts, 1.0, None)
    logits = jnp.dot(pooled, wlin_ref[...], preferred_element_type=jnp.float32)
    logits = logits + blin_ref[...]
    mx = jnp.max(logits, axis=1, keepdims=True)
    z = logits - mx
    out_ref[...] = z - jnp.log(jnp.sum(jnp.exp(z), axis=1, keepdims=True))


def kernel(x, edge_index, batch, Wl, bl, Wr, br, att, bias, Wlin, blin):
    loops = jnp.arange(N, dtype=edge_index.dtype)
    pad = jnp.full((EP - EDGES,), DUMMY, dtype=edge_index.dtype)
    src = jnp.concatenate([edge_index[0], loops, pad])
    dst = jnp.concatenate([edge_index[1], loops, pad])
    edges = jnp.stack([src, dst])
    x_pad = jnp.pad(x, ((0, NPAD - N), (0, 0)))

    xl, xr = pl.pallas_call(
        _xform_body,
        out_shape=(jax.ShapeDtypeStruct((NPAD, HID), jnp.float32),
                   jax.ShapeDtypeStruct((NPAD, HID), jnp.float32)),
    )(x_pad, Wl, bl.reshape(1, -1), Wr, br.reshape(1, -1))

    numer, denom = _sc_edge_call(xl, xr, edges, att)

    batch_pad = jnp.pad(batch, (0, NPAD - N), constant_values=NUM_GRAPHS)
    return pl.pallas_call(
        _final_body,
        out_shape=jax.ShapeDtypeStruct((NUM_GRAPHS, OUT_CH), jnp.float32),
    )(numer, denom, bias.reshape(1, -1), batch_pad.reshape(1, -1),
      Wlin, blin.reshape(1, -1))
